# baseline (speedup 1.0000x reference)
import functools

import jax
import jax.numpy as jnp
from jax import lax
from jax.experimental import pallas as pl
from jax.experimental.pallas import tpu as pltpu, tpu_sc as plsc

MODEL_DIM = 2048
NUM_EXPERTS = 16
K = 2
N_TOKENS = 16384
BLOCK_TOKENS = 1024


def _gate_kernel(x_ref, w_ref, b_ref, idx_ref, score_ref):
    x = x_ref[...]
    w = w_ref[...]
    logits = jax.lax.dot_general(
        w, x, (((1,), (1,)), ((), ())), preferred_element_type=jnp.float32
    )
    logits = logits + b_ref[...]

    iota = jax.lax.broadcasted_iota(jnp.int32, logits.shape, 0)
    big = jnp.int32(NUM_EXPERTS)

    m1 = jnp.max(logits, axis=0, keepdims=True)
    i1 = jnp.min(jnp.where(logits == m1, iota, big), axis=0, keepdims=True)
    masked = jnp.where(iota == i1, -jnp.inf, logits)
    m2 = jnp.max(masked, axis=0, keepdims=True)
    i2 = jnp.min(jnp.where(masked == m2, iota, big), axis=0, keepdims=True)

    e = jnp.exp(m2 - m1)
    denom = 1.0 + e
    s1 = 1.0 / denom
    s2 = e / denom

    idx_ref[...] = jnp.concatenate([i1, i2], axis=0)
    score_ref[...] = jnp.concatenate([s1, s2], axis=0)


@jax.jit
def kernel(x, W, b):
    n = x.shape[0]
    grid = (n // BLOCK_TOKENS,)
    b2 = b.reshape(NUM_EXPERTS, 1)
    idx_t, scores_t = pl.pallas_call(
        _gate_kernel,
        grid=grid,
        in_specs=[
            pl.BlockSpec((BLOCK_TOKENS, MODEL_DIM), lambda i: (i, 0)),
            pl.BlockSpec((NUM_EXPERTS, MODEL_DIM), lambda i: (0, 0)),
            pl.BlockSpec((NUM_EXPERTS, 1), lambda i: (0, 0)),
        ],
        out_specs=[
            pl.BlockSpec((K, BLOCK_TOKENS), lambda i: (0, i)),
            pl.BlockSpec((K, BLOCK_TOKENS), lambda i: (0, i)),
        ],
        out_shape=[
            jax.ShapeDtypeStruct((K, n), jnp.int32),
            jax.ShapeDtypeStruct((K, n), jnp.float32),
        ],
    )(x, W, b2)

    probe = _sc_probe(x)
    scores = scores_t.T + 0.0 * jnp.sum(probe)
    return (idx_t.T, scores)


_SC_CHUNK = 32


def _sc_probe_body(x_hbm, out_hbm, buf, sem):
    wid = lax.axis_index("s") * 2 + lax.axis_index("c")
    base = wid * _SC_CHUNK
    pltpu.async_copy(x_hbm.at[pl.ds(base, _SC_CHUNK)], buf, sem).wait()
    pltpu.sync_copy(buf.at[0, pl.ds(0, 16)], out_hbm.at[wid])


def _sc_probe(x):
    mesh = plsc.VectorSubcoreMesh(core_axis_name="c", subcore_axis_name="s")
    f = functools.partial(
        pl.kernel,
        mesh=mesh,
        out_type=jax.ShapeDtypeStruct((32, 16), jnp.float32),
        scratch_types=[
            pltpu.VMEM((_SC_CHUNK, MODEL_DIM), jnp.float32),
            pltpu.SemaphoreType.DMA,
        ],
    )(_sc_probe_body)
    return f(x)


# speedup vs baseline: 1.0058x; 1.0058x over previous
import functools

import jax
import jax.numpy as jnp
from jax import lax
from jax.experimental import pallas as pl
from jax.experimental.pallas import tpu as pltpu, tpu_sc as plsc

MODEL_DIM = 2048
NUM_EXPERTS = 16
K = 2
N_TOKENS = 16384
BLOCK_TOKENS = 1024


def _gate_kernel(x_ref, w_ref, b_ref, idx_ref, score_ref):
    x = x_ref[...]
    w = w_ref[...]
    logits = jax.lax.dot_general(
        w, x, (((1,), (1,)), ((), ())), preferred_element_type=jnp.float32
    )
    logits = logits + b_ref[...]

    iota = jax.lax.broadcasted_iota(jnp.int32, logits.shape, 0)
    big = jnp.int32(NUM_EXPERTS)

    m1 = jnp.max(logits, axis=0, keepdims=True)
    i1 = jnp.min(jnp.where(logits == m1, iota, big), axis=0, keepdims=True)
    masked = jnp.where(iota == i1, -jnp.inf, logits)
    m2 = jnp.max(masked, axis=0, keepdims=True)
    i2 = jnp.min(jnp.where(masked == m2, iota, big), axis=0, keepdims=True)

    e = jnp.exp(m2 - m1)
    denom = 1.0 + e
    s1 = 1.0 / denom
    s2 = e / denom

    idx_ref[...] = jnp.concatenate([i1, i2], axis=0)
    score_ref[...] = jnp.concatenate([s1, s2], axis=0)


@jax.jit
def kernel(x, W, b):
    n = x.shape[0]
    grid = (n // BLOCK_TOKENS,)
    b2 = b.reshape(NUM_EXPERTS, 1)
    probe = _sc_probe(x)
    idx_t, scores_t = pl.pallas_call(
        _gate_kernel,
        grid=grid,
        in_specs=[
            pl.BlockSpec((BLOCK_TOKENS, MODEL_DIM), lambda i: (i, 0)),
            pl.BlockSpec((NUM_EXPERTS, MODEL_DIM), lambda i: (0, 0)),
            pl.BlockSpec((NUM_EXPERTS, 1), lambda i: (0, 0)),
        ],
        out_specs=[
            pl.BlockSpec((K, BLOCK_TOKENS), lambda i: (0, i)),
            pl.BlockSpec((K, BLOCK_TOKENS), lambda i: (0, i)),
        ],
        out_shape=[
            jax.ShapeDtypeStruct((K, n), jnp.int32),
            jax.ShapeDtypeStruct((K, n), jnp.float32),
        ],
    )(x, W, b2)

    scores = scores_t.T + 0.0 * jnp.sum(probe)
    return (idx_t.T, scores)


_SC_CHUNK = 32


def _sc_probe_body(x_hbm, out_hbm, buf, sem):
    wid = lax.axis_index("s") * 2 + lax.axis_index("c")
    base = wid * _SC_CHUNK
    pltpu.async_copy(x_hbm.at[pl.ds(base, _SC_CHUNK)], buf, sem).wait()
    pltpu.sync_copy(buf.at[0, pl.ds(0, 16)], out_hbm.at[wid])


def _sc_probe(x):
    mesh = plsc.VectorSubcoreMesh(core_axis_name="c", subcore_axis_name="s")
    f = functools.partial(
        pl.kernel,
        mesh=mesh,
        out_type=jax.ShapeDtypeStruct((32, 16), jnp.float32),
        scratch_types=[
            pltpu.VMEM((_SC_CHUNK, MODEL_DIM), jnp.float32),
            pltpu.SemaphoreType.DMA,
        ],
    )(_sc_probe_body)
    return f(x)


# speedup vs baseline: 1.0431x; 1.0371x over previous
import functools

import jax
import jax.numpy as jnp
from jax import lax
from jax.experimental import pallas as pl
from jax.experimental.pallas import tpu as pltpu, tpu_sc as plsc

MODEL_DIM = 2048
NUM_EXPERTS = 16
K = 2
N_TOKENS = 16384
BLOCK_TOKENS = 1024

_NW = 32
_TPW = N_TOKENS // _NW


def _logits_kernel(x_ref, w_ref, b_ref, out_ref):
    x = x_ref[...]
    w = w_ref[...]
    logits = jax.lax.dot_general(
        w, x, (((1,), (1,)), ((), ())), preferred_element_type=jnp.float32
    )
    out_ref[...] = logits + b_ref[...]


def _route_body(logits_hbm, idx_hbm, score_hbm, lv, oi1, oi2, os1, os2, sem):
    wid = lax.axis_index("s") * 2 + lax.axis_index("c")
    base = wid * _TPW
    pltpu.async_copy(logits_hbm.at[:, pl.ds(base, _TPW)], lv, sem).wait()

    def step(j, _):
        col = j * 16
        v0 = lv[0, pl.ds(col, 16)]
        m1 = v0
        i1 = jnp.zeros((16,), jnp.int32)
        m2 = jnp.full((16,), -jnp.inf, jnp.float32)
        i2 = jnp.zeros((16,), jnp.int32)
        for e in range(1, NUM_EXPERTS):
            ve = lv[e, pl.ds(col, 16)]
            ei = jnp.full((16,), e, jnp.int32)
            gt1 = ve > m1
            gt2 = ve > m2
            i2 = jnp.where(gt1, i1, jnp.where(gt2, ei, i2))
            m2 = jnp.where(gt1, m1, jnp.where(gt2, ve, m2))
            i1 = jnp.where(gt1, ei, i1)
            m1 = jnp.where(gt1, ve, m1)
        ex = jnp.exp(m2 - m1)
        den = 1.0 + ex
        oi1[pl.ds(col, 16)] = i1
        oi2[pl.ds(col, 16)] = i2
        os1[pl.ds(col, 16)] = 1.0 / den
        os2[pl.ds(col, 16)] = ex / den
        return 0

    lax.fori_loop(0, _TPW // 16, step, 0)

    pltpu.sync_copy(oi1, idx_hbm.at[0, pl.ds(base, _TPW)])
    pltpu.sync_copy(oi2, idx_hbm.at[1, pl.ds(base, _TPW)])
    pltpu.sync_copy(os1, score_hbm.at[0, pl.ds(base, _TPW)])
    pltpu.sync_copy(os2, score_hbm.at[1, pl.ds(base, _TPW)])


def _sc_route(logits_t):
    mesh = plsc.VectorSubcoreMesh(core_axis_name="c", subcore_axis_name="s")
    f = functools.partial(
        pl.kernel,
        mesh=mesh,
        out_type=[
            jax.ShapeDtypeStruct((K, N_TOKENS), jnp.int32),
            jax.ShapeDtypeStruct((K, N_TOKENS), jnp.float32),
        ],
        scratch_types=[
            pltpu.VMEM((NUM_EXPERTS, _TPW), jnp.float32),
            pltpu.VMEM((_TPW,), jnp.int32),
            pltpu.VMEM((_TPW,), jnp.int32),
            pltpu.VMEM((_TPW,), jnp.float32),
            pltpu.VMEM((_TPW,), jnp.float32),
            pltpu.SemaphoreType.DMA,
        ],
    )(_route_body)
    return f(logits_t)


@jax.jit
def kernel(x, W, b):
    n = x.shape[0]
    grid = (n // BLOCK_TOKENS,)
    b2 = b.reshape(NUM_EXPERTS, 1)
    logits_t = pl.pallas_call(
        _logits_kernel,
        grid=grid,
        in_specs=[
            pl.BlockSpec((BLOCK_TOKENS, MODEL_DIM), lambda i: (i, 0)),
            pl.BlockSpec((NUM_EXPERTS, MODEL_DIM), lambda i: (0, 0)),
            pl.BlockSpec((NUM_EXPERTS, 1), lambda i: (0, 0)),
        ],
        out_specs=pl.BlockSpec((NUM_EXPERTS, BLOCK_TOKENS), lambda i: (0, i)),
        out_shape=jax.ShapeDtypeStruct((NUM_EXPERTS, n), jnp.float32),
    )(x, W, b2)
    idx_t, scores_t = _sc_route(logits_t)
    return (idx_t.T, scores_t.T)


# speedup vs baseline: 1.5040x; 1.4418x over previous
import functools

import jax
import jax.numpy as jnp
from jax.experimental import pallas as pl
from jax.experimental.pallas import tpu as pltpu

MODEL_DIM = 2048
NUM_EXPERTS = 16
K = 2
N_TOKENS = 16384
BLOCK_TOKENS = 1024


def _gate_kernel(x_ref, w_ref, b_ref, idx_ref, score_ref):
    x = x_ref[...]
    w = w_ref[...]
    logits = jax.lax.dot_general(
        w, x, (((1,), (1,)), ((), ())), preferred_element_type=jnp.float32
    )
    logits = logits + b_ref[...]

    iota = jax.lax.broadcasted_iota(jnp.int32, logits.shape, 0)
    big = jnp.int32(NUM_EXPERTS)

    m1 = jnp.max(logits, axis=0, keepdims=True)
    i1 = jnp.min(jnp.where(logits == m1, iota, big), axis=0, keepdims=True)
    masked = jnp.where(iota == i1, -jnp.inf, logits)
    m2 = jnp.max(masked, axis=0, keepdims=True)
    i2 = jnp.min(jnp.where(masked == m2, iota, big), axis=0, keepdims=True)

    e = jnp.exp(m2 - m1)
    denom = 1.0 + e
    s1 = 1.0 / denom
    s2 = e / denom

    idx_ref[...] = jnp.concatenate([i1, i2], axis=0)
    score_ref[...] = jnp.concatenate([s1, s2], axis=0)


@jax.jit
def kernel(x, W, b):
    n = x.shape[0]
    grid = (n // BLOCK_TOKENS,)
    b2 = b.reshape(NUM_EXPERTS, 1)
    idx_t, scores_t = pl.pallas_call(
        _gate_kernel,
        grid=grid,
        in_specs=[
            pl.BlockSpec((BLOCK_TOKENS, MODEL_DIM), lambda i: (i, 0)),
            pl.BlockSpec((NUM_EXPERTS, MODEL_DIM), lambda i: (0, 0)),
            pl.BlockSpec((NUM_EXPERTS, 1), lambda i: (0, 0)),
        ],
        out_specs=[
            pl.BlockSpec((K, BLOCK_TOKENS), lambda i: (0, i)),
            pl.BlockSpec((K, BLOCK_TOKENS), lambda i: (0, i)),
        ],
        out_shape=[
            jax.ShapeDtypeStruct((K, n), jnp.int32),
            jax.ShapeDtypeStruct((K, n), jnp.float32),
        ],
        compiler_params=pltpu.CompilerParams(
            dimension_semantics=("parallel",)
        ),
    )(x, W, b2)
    return (idx_t.T, scores_t.T)


# speedup vs baseline: 1.5058x; 1.0012x over previous
import functools

import jax
import jax.numpy as jnp
from jax.experimental import pallas as pl

MODEL_DIM = 2048
NUM_EXPERTS = 16
K = 2
N_TOKENS = 16384
BLOCK_TOKENS = 1024


def _gate_kernel(x_ref, w_ref, b_ref, idx_ref, score_ref):
    x = x_ref[...]
    w = w_ref[...]
    logits = jax.lax.dot_general(
        w, x, (((1,), (1,)), ((), ())), preferred_element_type=jnp.float32
    )
    logits = logits + b_ref[...]

    iota = jax.lax.broadcasted_iota(jnp.int32, logits.shape, 0)
    big = jnp.int32(NUM_EXPERTS)

    m1 = jnp.max(logits, axis=0, keepdims=True)
    i1 = jnp.min(jnp.where(logits == m1, iota, big), axis=0, keepdims=True)
    masked = jnp.where(iota == i1, -jnp.inf, logits)
    m2 = jnp.max(masked, axis=0, keepdims=True)
    i2 = jnp.min(jnp.where(masked == m2, iota, big), axis=0, keepdims=True)

    e = jnp.exp(m2 - m1)
    denom = 1.0 + e
    s1 = 1.0 / denom
    s2 = e / denom

    idx_ref[...] = jnp.concatenate([i1, i2], axis=0)
    score_ref[...] = jnp.concatenate([s1, s2], axis=0)


@jax.jit
def kernel(x, W, b):
    n = x.shape[0]
    grid = (n // BLOCK_TOKENS,)
    b2 = b.reshape(NUM_EXPERTS, 1)
    idx_t, scores_t = pl.pallas_call(
        _gate_kernel,
        grid=grid,
        in_specs=[
            pl.BlockSpec((BLOCK_TOKENS, MODEL_DIM), lambda i: (i, 0)),
            pl.BlockSpec((NUM_EXPERTS, MODEL_DIM), lambda i: (0, 0)),
            pl.BlockSpec((NUM_EXPERTS, 1), lambda i: (0, 0)),
        ],
        out_specs=[
            pl.BlockSpec((K, BLOCK_TOKENS), lambda i: (0, i)),
            pl.BlockSpec((K, BLOCK_TOKENS), lambda i: (0, i)),
        ],
        out_shape=[
            jax.ShapeDtypeStruct((K, n), jnp.int32),
            jax.ShapeDtypeStruct((K, n), jnp.float32),
        ],
    )(x, W, b2)
    return (idx_t.T, scores_t.T)
